# baseline (speedup 1.0000x reference)
import functools

import numpy as np
import jax
from jax import lax
import jax.numpy as jnp
from jax.experimental import pallas as pl
from jax.experimental.pallas import tpu as pltpu
from jax.experimental.pallas import tpu_sc as plsc

_GAMMA = 0.5
_UBLK = 40


def _argmin_kernel(d2_ref, idx_ref):
    d2 = d2_ref[0]
    bb, n = d2.shape
    rowmin = jnp.min(d2, axis=1, keepdims=True)
    iota = jax.lax.broadcasted_iota(jnp.int32, (bb, n), 1)
    idx = jnp.min(jnp.where(d2 == rowmin, iota, n), axis=1)
    idx_ref[0, 0, :] = idx


def _expand_kernel(h_ref, x_ref, lm_ref, e_ref, t_ref, out_ref):
    w = e_ref.shape[1]
    ub = e_ref.shape[0]
    f = t_ref.shape[0]
    n = lm_ref.shape[1] // f
    nchunks = n // ub

    ht = jnp.exp(-_GAMMA * jnp.transpose(h_ref[...], (1, 0))[:n, :])
    xt = jax.lax.dot_general(
        x_ref[...], t_ref[...],
        dimension_numbers=(((1,), (0,)), ((), ())),
        preferred_element_type=jnp.float32,
    )
    for k in range(nchunks):
        h_rep = jax.lax.dot_general(
            ht[k * ub:(k + 1) * ub, :], e_ref[...],
            dimension_numbers=(((0,), (0,)), ((), ())),
            preferred_element_type=jnp.float32,
        )
        out_ref[:, k * w:(k + 1) * w] = h_rep * (xt - lm_ref[0, k * w:(k + 1) * w][None, :])


def _sc_gather(qd, idx, b, n):
    info = plsc.get_sparse_core_info()
    nc, ns = info.num_cores, info.num_subcores
    nw = nc * ns
    b_per_w = b // nw
    mesh = plsc.VectorSubcoreMesh(core_axis_name="c", subcore_axis_name="s")

    @functools.partial(
        pl.kernel, mesh=mesh,
        out_type=jax.ShapeDtypeStruct((b, n), jnp.float32),
        scratch_types=[
            pltpu.VMEM((b_per_w,), jnp.int32),
            pltpu.VMEM((b_per_w, n), jnp.float32),
            pltpu.SemaphoreType.DMA,
        ],
    )
    def gather_k(qd_hbm, idx_hbm, out_hbm, idx_v, rows_v, sem):
        wid = lax.axis_index("s") * nc + lax.axis_index("c")
        base = wid * b_per_w
        pltpu.sync_copy(idx_hbm.at[pl.ds(base, b_per_w)], idx_v)
        pltpu.async_copy(qd_hbm.at[idx_v], rows_v, sem).wait()
        pltpu.sync_copy(rows_v, out_hbm.at[pl.ds(base, b_per_w)])

    return gather_k(qd, idx)


@jax.jit
def kernel(x, d2, qd, landmarks):
    b, f = x.shape
    n = qd.shape[0]
    ub = _UBLK
    w = ub * f

    bb = 128
    nblk = b // bb
    idx2d = pl.pallas_call(
        _argmin_kernel,
        grid=(nblk,),
        in_specs=[pl.BlockSpec((1, bb, n), lambda i: (i, 0, 0))],
        out_specs=pl.BlockSpec((1, 1, bb), lambda i: (i, 0, 0)),
        out_shape=jax.ShapeDtypeStruct((nblk, 1, bb), jnp.int32),
    )(d2.reshape(nblk, bb, n))
    idx = idx2d.reshape(b)

    npad = 1280
    qd_pad = jnp.pad(qd, ((0, 0), (0, npad - n)))
    h_raw = _sc_gather(qd_pad, idx, b, npad)

    lanes = np.arange(w)
    e_mat = jnp.asarray((lanes[None, :] // f) == np.arange(ub)[:, None],
                        dtype=jnp.float32)
    t_mat = jnp.asarray((lanes[None, :] % f) == np.arange(f)[:, None],
                        dtype=jnp.float32)
    lm_flat = landmarks.reshape(1, n * f)

    out_flat = pl.pallas_call(
        _expand_kernel,
        grid=(nblk,),
        in_specs=[
            pl.BlockSpec((bb, npad), lambda i: (i, 0)),
            pl.BlockSpec((bb, f), lambda i: (i, 0)),
            pl.BlockSpec((1, n * f), lambda i: (0, 0)),
            pl.BlockSpec((ub, w), lambda i: (0, 0)),
            pl.BlockSpec((f, w), lambda i: (0, 0)),
        ],
        out_specs=pl.BlockSpec((bb, n * f), lambda i: (i, 0)),
        out_shape=jax.ShapeDtypeStruct((b, n * f), jnp.float32),
        compiler_params=pltpu.CompilerParams(
            dimension_semantics=("parallel",),
        ),
    )(h_raw, x, lm_flat, e_mat, t_mat)

    return out_flat.reshape(b, n, f)
